# baseline (speedup 1.0000x reference)
import functools

import jax
import jax.numpy as jnp
from jax import lax
from jax.experimental import pallas as pl
from jax.experimental.pallas import tpu as pltpu
from jax.experimental.pallas import tpu_sc as plsc

D = 128
L = 16
NC = 2
NS = 16
C = 128
DEGW = 16
IGN = -1


def _mesh():
    return plsc.VectorSubcoreMesh(
        core_axis_name="c", subcore_axis_name="s",
        num_cores=NC, num_subcores=NS)


def _make_deg_kernel(NP, CPW):
    rows_per_tile = NP // NS

    @functools.partial(
        pl.kernel,
        out_type=jax.ShapeDtypeStruct((NC * NP, DEGW), jnp.float32),
        mesh=_mesh(),
        scratch_types=[
            pltpu.VMEM((2 * CPW, C), jnp.int32),
            pltpu.VMEM((C, DEGW), jnp.float32),
            pltpu.VMEM((rows_per_tile, DEGW), jnp.float32),
            pltpu.VMEM_SHARED((NC * NP, DEGW), jnp.float32),
        ],
        compiler_params=pltpu.CompilerParams(use_tc_tiling_on_sc=False),
    )
    def deg_kernel(idx_hbm, deg_hbm, idx_v, ones_v, z_v, deg_sh):
        c = lax.axis_index("c")
        s = lax.axis_index("s")

        one16 = jnp.full((L,), 1.0, jnp.float32)
        zero16 = jnp.zeros((L,), jnp.float32)

        def fill_ones(r, _):
            ones_v[r, :] = one16
            return 0
        lax.fori_loop(0, C, fill_ones, 0)

        def fill_zeros(r, _):
            z_v[r, :] = zero16
            return 0
        lax.fori_loop(0, rows_per_tile, fill_zeros, 0)

        pltpu.sync_copy(z_v, deg_sh.at[pl.ds(s * rows_per_tile,
                                             rows_per_tile)])
        pltpu.sync_copy(z_v, deg_sh.at[pl.ds(NP + s * rows_per_tile,
                                             rows_per_tile)])
        plsc.subcore_barrier()

        pltpu.sync_copy(idx_hbm.at[s], idx_v)

        def body(i, _):
            pltpu.sync_copy(ones_v, deg_sh.at[idx_v.at[i]], add=True)
            return 0
        lax.fori_loop(0, 2 * CPW, body, 0)

        plsc.subcore_barrier()
        row0 = c * NP + s * rows_per_tile
        pltpu.sync_copy(deg_sh.at[pl.ds(row0, rows_per_tile)],
                        deg_hbm.at[pl.ds(row0, rows_per_tile)])

    return deg_kernel


def _make_edge_kernel(NP, CPW):
    half = NP // NC
    rows_per_tile = half // NS

    @functools.partial(
        pl.kernel,
        out_type=jax.ShapeDtypeStruct((NP, D), jnp.float32),
        mesh=_mesh(),
        scratch_types=[
            pltpu.VMEM((CPW, C), jnp.int32),
            pltpu.VMEM((CPW, C), jnp.int32),
            pltpu.VMEM((2, C, D), jnp.float32),
            pltpu.VMEM_SHARED((half, D), jnp.float32),
            pltpu.SemaphoreType.DMA((2,)),
        ],
        compiler_params=pltpu.CompilerParams(use_tc_tiling_on_sc=False),
    )
    def edge_kernel(h_hbm, src_hbm, dst_hbm, agg_hbm,
                    src_v, dst_v, rows_v, agg_sh, sem):
        c = lax.axis_index("c")
        s = lax.axis_index("s")
        row0 = s * rows_per_tile

        zero16 = jnp.zeros((L,), jnp.float32)

        def fill_zeros(r, _):
            for k in range(D // L):
                rows_v[0, r, pl.ds(k * L, L)] = zero16
            return 0
        lax.fori_loop(0, C, fill_zeros, 0)

        zbuf = rows_v.at[0]
        nfull = rows_per_tile // C
        rem = rows_per_tile - nfull * C
        for k in range(nfull):
            pltpu.sync_copy(zbuf, agg_sh.at[pl.ds(row0 + k * C, C)])
        if rem:
            pltpu.sync_copy(zbuf.at[pl.ds(0, rem)],
                            agg_sh.at[pl.ds(row0 + nfull * C, rem)])
        plsc.subcore_barrier()

        pltpu.sync_copy(src_hbm.at[c, s], src_v)
        pltpu.sync_copy(dst_hbm.at[c, s], dst_v)

        def gather(i, b):
            pltpu.async_copy(
                h_hbm.at[plsc.Indices(src_v.at[i], ignored_value=IGN)],
                rows_v.at[b], sem.at[b])

        gather(0, 0)

        def body(i, _):
            b = lax.rem(i, 2)
            nb = 1 - b

            @pl.when(i + 1 < CPW)
            def _():
                gather(i + 1, nb)

            pltpu.make_async_copy(
                h_hbm.at[plsc.Indices(src_v.at[i], ignored_value=IGN)],
                rows_v.at[b], sem.at[b]).wait()
            pltpu.sync_copy(
                rows_v.at[b],
                agg_sh.at[plsc.Indices(dst_v.at[i], ignored_value=IGN)],
                add=True)
            return 0
        lax.fori_loop(0, CPW, body, 0)

        plsc.subcore_barrier()
        out0 = pl.multiple_of(c * half + row0, 8)
        pltpu.sync_copy(agg_sh.at[pl.ds(row0, rows_per_tile)],
                        agg_hbm.at[pl.ds(out0, rows_per_tile)])

    return edge_kernel


def _xs_body(N, x_ref, deg_ref, o_ref):
    NP = x_ref.shape[0]
    dego = deg_ref[pl.ds(0, NP), :]
    norm = lax.rsqrt(jnp.maximum(dego[:, 0:1], 1.0))
    o_ref[...] = x_ref[...] * norm


def _layer_body(N, agg_ref, deg_ref, w_ref, b_ref, o_ref):
    NP = agg_ref.shape[0]
    degi = deg_ref[pl.ds(NP, NP), :]
    dego = deg_ref[pl.ds(0, NP), :]
    ni = lax.rsqrt(jnp.maximum(degi[:, 0:1], 1.0))
    no = lax.rsqrt(jnp.maximum(dego[:, 0:1], 1.0))
    h = jnp.dot(agg_ref[...] * ni, w_ref[...],
                preferred_element_type=jnp.float32)
    h = jnp.maximum(h + b_ref[...], 0.0)
    mask = lax.broadcasted_iota(jnp.int32, (NP, 1), 0) < N
    o_ref[...] = jnp.where(mask, h * no, 0.0)


def _final_body(N, agg_ref, deg_ref, w_ref, b_ref,
                wd1_ref, bd1_ref, wd2_ref, bd2_ref, o_ref):
    NP = agg_ref.shape[0]
    degi = deg_ref[pl.ds(NP, NP), :]
    ni = lax.rsqrt(jnp.maximum(degi[:, 0:1], 1.0))
    h = jnp.dot(agg_ref[...] * ni, w_ref[...],
                preferred_element_type=jnp.float32)
    h = jnp.maximum(h + b_ref[...], 0.0)
    mask = lax.broadcasted_iota(jnp.int32, (NP, 1), 0) < N
    h = jnp.where(mask, h, 0.0)
    g = jnp.sum(h, axis=0, keepdims=True)
    g = jnp.dot(g, wd1_ref[...], preferred_element_type=jnp.float32)
    g = jnp.maximum(g + bd1_ref[...], 0.0)
    g = jnp.dot(g, wd2_ref[...], preferred_element_type=jnp.float32)
    o_ref[...] = g + bd2_ref[...]


def kernel(x, edge_index, W0, b0, W1, b1, Wd1, bd1, Wd2, bd2):
    N = x.shape[0]
    E = edge_index.shape[1]
    NP = ((N + 2 * D) // (2 * D)) * (2 * D)
    half = NP // NC
    CPW = -(-E // (NS * C))
    EP = NS * CPW * C

    src = edge_index[0]
    dst = edge_index[1]

    npad_rows = NP - N
    pad = N + (jnp.arange(EP - E, dtype=jnp.int32) % npad_rows)
    srcp = jnp.concatenate([src, pad]).reshape(NS, CPW, C)
    dstp = jnp.concatenate([dst, pad]).reshape(NS, CPW, C)
    degidx = jnp.concatenate([srcp, dstp + NP], axis=1)

    padi = jnp.full((EP - E,), IGN, jnp.int32)
    planes_src, planes_dst = [], []
    for ci in range(NC):
        mine = (dst // half) == ci
        planes_src.append(
            jnp.concatenate([jnp.where(mine, src, IGN), padi])
            .reshape(NS, CPW, C))
        planes_dst.append(
            jnp.concatenate([jnp.where(mine, dst - ci * half, IGN), padi])
            .reshape(NS, CPW, C))
    src2 = jnp.stack(planes_src)
    dst2 = jnp.stack(planes_dst)

    xp = jnp.pad(x, ((0, NP - N), (0, 0)))
    b0r, b1r = b0.reshape(1, D), b1.reshape(1, D)
    bd1r, bd2r = bd1.reshape(1, D), bd2.reshape(1, D)

    deg_kernel = _make_deg_kernel(NP, CPW)
    edge_kernel = _make_edge_kernel(NP, CPW)

    deg = deg_kernel(degidx)

    xs = pl.pallas_call(
        functools.partial(_xs_body, N),
        out_shape=jax.ShapeDtypeStruct((NP, D), jnp.float32),
    )(xp, deg)

    agg1 = edge_kernel(xs, src2, dst2)

    h1s = pl.pallas_call(
        functools.partial(_layer_body, N),
        out_shape=jax.ShapeDtypeStruct((NP, D), jnp.float32),
    )(agg1, deg, W0, b0r)

    agg2 = edge_kernel(h1s, src2, dst2)

    out = pl.pallas_call(
        functools.partial(_final_body, N),
        out_shape=jax.ShapeDtypeStruct((1, D), jnp.float32),
    )(agg2, deg, W1, b1r, Wd1, bd1r, Wd2, bd2r)

    return out


# speedup vs baseline: 1.3390x; 1.3390x over previous
import functools

import jax
import jax.numpy as jnp
from jax import lax
from jax.experimental import pallas as pl
from jax.experimental.pallas import tpu as pltpu
from jax.experimental.pallas import tpu_sc as plsc

D = 128
L = 16
NC = 2
NS = 16
C = 128
DEGW = 16
IGN = -1


def _mesh():
    return plsc.VectorSubcoreMesh(
        core_axis_name="c", subcore_axis_name="s",
        num_cores=NC, num_subcores=NS)


def _make_deg_kernel(NP, CPW):
    rows_per_tile = NP // NS

    @functools.partial(
        pl.kernel,
        out_type=jax.ShapeDtypeStruct((NC * NP, DEGW), jnp.float32),
        mesh=_mesh(),
        scratch_types=[
            pltpu.VMEM((2 * CPW, C), jnp.int32),
            pltpu.VMEM((C, DEGW), jnp.float32),
            pltpu.VMEM((rows_per_tile, DEGW), jnp.float32),
            pltpu.VMEM_SHARED((NC * NP, DEGW), jnp.float32),
        ],
        compiler_params=pltpu.CompilerParams(use_tc_tiling_on_sc=False),
    )
    def deg_kernel(idx_hbm, deg_hbm, idx_v, ones_v, z_v, deg_sh):
        c = lax.axis_index("c")
        s = lax.axis_index("s")

        one16 = jnp.full((L,), 1.0, jnp.float32)
        zero16 = jnp.zeros((L,), jnp.float32)

        def fill_ones(r, _):
            ones_v[r, :] = one16
            return 0
        lax.fori_loop(0, C, fill_ones, 0)

        def fill_zeros(r, _):
            z_v[r, :] = zero16
            return 0
        lax.fori_loop(0, rows_per_tile, fill_zeros, 0)

        pltpu.sync_copy(z_v, deg_sh.at[pl.ds(s * rows_per_tile,
                                             rows_per_tile)])
        pltpu.sync_copy(z_v, deg_sh.at[pl.ds(NP + s * rows_per_tile,
                                             rows_per_tile)])
        plsc.subcore_barrier()

        pltpu.sync_copy(idx_hbm.at[s], idx_v)

        def body(i, _):
            pltpu.sync_copy(ones_v, deg_sh.at[idx_v.at[i]], add=True)
            return 0
        lax.fori_loop(c * CPW, (c + 1) * CPW, body, 0)

        plsc.subcore_barrier()
        row0 = c * NP + s * rows_per_tile
        pltpu.sync_copy(deg_sh.at[pl.ds(row0, rows_per_tile)],
                        deg_hbm.at[pl.ds(row0, rows_per_tile)])

    return deg_kernel


def _make_edge_kernel(NP, CPW):
    DH = D // NC
    rows_per_tile = NP // NS
    NB = 3

    @functools.partial(
        pl.kernel,
        out_type=jax.ShapeDtypeStruct((NC * NP, DH), jnp.float32),
        mesh=_mesh(),
        scratch_types=[
            pltpu.VMEM((CPW, C), jnp.int32),
            pltpu.VMEM((CPW, C), jnp.int32),
            pltpu.VMEM((NB, C, DH), jnp.float32),
            pltpu.VMEM_SHARED((NP, DH), jnp.float32),
            pltpu.SemaphoreType.DMA((NB,)),
        ],
        compiler_params=pltpu.CompilerParams(use_tc_tiling_on_sc=False),
    )
    def edge_kernel(h_hbm, src_hbm, dst_hbm, agg_hbm,
                    src_v, dst_v, rows_v, agg_sh, sem):
        c = lax.axis_index("c")
        s = lax.axis_index("s")
        row0 = s * rows_per_tile

        zero16 = jnp.zeros((L,), jnp.float32)

        def fill_zeros(r, _):
            for k in range(DH // L):
                rows_v[0, r, pl.ds(k * L, L)] = zero16
            return 0
        lax.fori_loop(0, C, fill_zeros, 0)

        zbuf = rows_v.at[0]
        nfull = rows_per_tile // C
        rem = rows_per_tile - nfull * C
        for k in range(nfull):
            pltpu.sync_copy(zbuf, agg_sh.at[pl.ds(row0 + k * C, C)])
        if rem:
            pltpu.sync_copy(zbuf.at[pl.ds(0, rem)],
                            agg_sh.at[pl.ds(row0 + nfull * C, rem)])
        plsc.subcore_barrier()

        pltpu.sync_copy(src_hbm.at[c, s], src_v)
        pltpu.sync_copy(dst_hbm.at[s], dst_v)

        def gather(i, b):
            pltpu.async_copy(h_hbm.at[src_v.at[i]], rows_v.at[b], sem.at[b])

        gather(0, 0)
        gather(1, 1)

        def body(i, _):
            b = lax.rem(i, NB)

            @pl.when(i + 2 < CPW)
            def _():
                gather(i + 2, lax.rem(i + 2, NB))

            pltpu.make_async_copy(h_hbm.at[src_v.at[i]], rows_v.at[b],
                                  sem.at[b]).wait()
            pltpu.sync_copy(rows_v.at[b], agg_sh.at[dst_v.at[i]], add=True)
            return 0
        lax.fori_loop(0, CPW, body, 0)

        plsc.subcore_barrier()
        pltpu.sync_copy(agg_sh.at[pl.ds(row0, rows_per_tile)],
                        agg_hbm.at[pl.ds(c * NP + row0, rows_per_tile)])

    return edge_kernel


def _xs_body(N, x_ref, deg_ref, o_ref):
    NP = x_ref.shape[0]
    DH = D // NC
    dego = deg_ref[pl.ds(0, NP), :]
    norm = lax.rsqrt(jnp.maximum(dego[:, 0:1], 1.0))
    xs = x_ref[...] * norm
    o_ref[pl.ds(0, NP), :] = xs[:, :DH]
    o_ref[pl.ds(NP, NP), :] = xs[:, DH:]


def _layer_body(N, agg_ref, deg_ref, w_ref, b_ref, o_ref):
    NP = agg_ref.shape[0] // 2
    DH = D // NC
    agg = jnp.concatenate(
        [agg_ref[pl.ds(0, NP), :], agg_ref[pl.ds(NP, NP), :]], axis=1)
    degi = deg_ref[pl.ds(NP, NP), :]
    dego = deg_ref[pl.ds(0, NP), :]
    ni = lax.rsqrt(jnp.maximum(degi[:, 0:1], 1.0))
    no = lax.rsqrt(jnp.maximum(dego[:, 0:1], 1.0))
    h = jnp.dot(agg * ni, w_ref[...], preferred_element_type=jnp.float32)
    h = jnp.maximum(h + b_ref[...], 0.0)
    mask = lax.broadcasted_iota(jnp.int32, (NP, 1), 0) < N
    h = jnp.where(mask, h * no, 0.0)
    o_ref[pl.ds(0, NP), :] = h[:, :DH]
    o_ref[pl.ds(NP, NP), :] = h[:, DH:]


def _final_body(N, agg_ref, deg_ref, w_ref, b_ref,
                wd1_ref, bd1_ref, wd2_ref, bd2_ref, o_ref):
    NP = agg_ref.shape[0] // 2
    agg = jnp.concatenate(
        [agg_ref[pl.ds(0, NP), :], agg_ref[pl.ds(NP, NP), :]], axis=1)
    degi = deg_ref[pl.ds(NP, NP), :]
    ni = lax.rsqrt(jnp.maximum(degi[:, 0:1], 1.0))
    h = jnp.dot(agg * ni, w_ref[...], preferred_element_type=jnp.float32)
    h = jnp.maximum(h + b_ref[...], 0.0)
    mask = lax.broadcasted_iota(jnp.int32, (NP, 1), 0) < N
    h = jnp.where(mask, h, 0.0)
    g = jnp.sum(h, axis=0, keepdims=True)
    g = jnp.dot(g, wd1_ref[...], preferred_element_type=jnp.float32)
    g = jnp.maximum(g + bd1_ref[...], 0.0)
    g = jnp.dot(g, wd2_ref[...], preferred_element_type=jnp.float32)
    o_ref[...] = g + bd2_ref[...]


def kernel(x, edge_index, W0, b0, W1, b1, Wd1, bd1, Wd2, bd2):
    N = x.shape[0]
    E = edge_index.shape[1]
    NP = ((N + 2 * D) // (2 * D)) * (2 * D)
    CPW = -(-E // (NS * C))
    EP = NS * CPW * C

    src = edge_index[0]
    dst = edge_index[1]

    npad_rows = NP - N
    pad = N + (jnp.arange(EP - E, dtype=jnp.int32) % npad_rows)
    srcp = jnp.concatenate([src, pad]).reshape(NS, CPW, C)
    dstp = jnp.concatenate([dst, pad]).reshape(NS, CPW, C)
    degidx = jnp.concatenate([srcp, dstp + NP], axis=1)
    src2 = jnp.stack([srcp, srcp + NP])

    xp = jnp.pad(x, ((0, NP - N), (0, 0)))
    b0r, b1r = b0.reshape(1, D), b1.reshape(1, D)
    bd1r, bd2r = bd1.reshape(1, D), bd2.reshape(1, D)

    deg_kernel = _make_deg_kernel(NP, CPW)
    edge_kernel = _make_edge_kernel(NP, CPW)
    DH = D // NC

    deg = deg_kernel(degidx)

    xs = pl.pallas_call(
        functools.partial(_xs_body, N),
        out_shape=jax.ShapeDtypeStruct((NC * NP, DH), jnp.float32),
    )(xp, deg)

    agg1 = edge_kernel(xs, src2, dstp)

    h1s = pl.pallas_call(
        functools.partial(_layer_body, N),
        out_shape=jax.ShapeDtypeStruct((NC * NP, DH), jnp.float32),
    )(agg1, deg, W0, b0r)

    agg2 = edge_kernel(h1s, src2, dstp)

    out = pl.pallas_call(
        functools.partial(_final_body, N),
        out_shape=jax.ShapeDtypeStruct((1, D), jnp.float32),
    )(agg2, deg, W1, b1r, Wd1, bd1r, Wd2, bd2r)

    return out


# speedup vs baseline: 1.4405x; 1.0758x over previous
import functools

import jax
import jax.numpy as jnp
from jax import lax
from jax.experimental import pallas as pl
from jax.experimental.pallas import tpu as pltpu
from jax.experimental.pallas import tpu_sc as plsc

D = 128
L = 16
NC = 2
NS = 16
C = 128
DEGW = 16
IGN = -1


def _mesh():
    return plsc.VectorSubcoreMesh(
        core_axis_name="c", subcore_axis_name="s",
        num_cores=NC, num_subcores=NS)


def _make_deg_kernel(NP, CPW):
    rows_per_tile = NP // NS

    @functools.partial(
        pl.kernel,
        out_type=jax.ShapeDtypeStruct((NC * NP, DEGW), jnp.float32),
        mesh=_mesh(),
        scratch_types=[
            pltpu.VMEM((2 * CPW, C), jnp.int32),
            pltpu.VMEM((C, DEGW), jnp.float32),
            pltpu.VMEM((rows_per_tile, DEGW), jnp.float32),
            pltpu.VMEM_SHARED((NC * NP, DEGW), jnp.float32),
        ],
        compiler_params=pltpu.CompilerParams(use_tc_tiling_on_sc=False),
    )
    def deg_kernel(idx_hbm, deg_hbm, idx_v, ones_v, z_v, deg_sh):
        c = lax.axis_index("c")
        s = lax.axis_index("s")

        one16 = jnp.full((L,), 1.0, jnp.float32)
        zero16 = jnp.zeros((L,), jnp.float32)

        def fill_ones(r, _):
            ones_v[r, :] = one16
            return 0
        lax.fori_loop(0, C, fill_ones, 0)

        def fill_zeros(r, _):
            z_v[r, :] = zero16
            return 0
        lax.fori_loop(0, rows_per_tile, fill_zeros, 0)

        pltpu.sync_copy(z_v, deg_sh.at[pl.ds(s * rows_per_tile,
                                             rows_per_tile)])
        pltpu.sync_copy(z_v, deg_sh.at[pl.ds(NP + s * rows_per_tile,
                                             rows_per_tile)])
        plsc.subcore_barrier()

        pltpu.sync_copy(idx_hbm.at[s], idx_v)

        def body(i, _):
            pltpu.sync_copy(ones_v, deg_sh.at[idx_v.at[i]], add=True)
            return 0
        lax.fori_loop(c * CPW, (c + 1) * CPW, body, 0)

        plsc.subcore_barrier()
        row0 = c * NP + s * rows_per_tile
        pltpu.sync_copy(deg_sh.at[pl.ds(row0, rows_per_tile)],
                        deg_hbm.at[pl.ds(row0, rows_per_tile)])

    return deg_kernel


def _make_edge_kernel(NP, CPW):
    DH = D // NC
    rows_per_tile = NP // NS
    NB = 3

    @functools.partial(
        pl.kernel,
        out_type=jax.ShapeDtypeStruct((NP, D), jnp.float32),
        mesh=_mesh(),
        scratch_types=[
            pltpu.VMEM((CPW, C), jnp.int32),
            pltpu.VMEM((CPW, C), jnp.int32),
            pltpu.VMEM((NB, C, DH), jnp.float32),
            pltpu.VMEM_SHARED((NP, DH), jnp.float32),
            pltpu.SemaphoreType.DMA((NB,)),
        ],
        compiler_params=pltpu.CompilerParams(use_tc_tiling_on_sc=False),
    )
    def edge_kernel(h_hbm, src_hbm, dst_hbm, agg_hbm,
                    src_v, dst_v, rows_v, agg_sh, sem):
        c = lax.axis_index("c")
        s = lax.axis_index("s")
        row0 = s * rows_per_tile

        zero16 = jnp.zeros((L,), jnp.float32)

        def fill_zeros(r, _):
            for k in range(DH // L):
                rows_v[0, r, pl.ds(k * L, L)] = zero16
            return 0
        lax.fori_loop(0, C, fill_zeros, 0)

        zbuf = rows_v.at[0]
        nfull = rows_per_tile // C
        rem = rows_per_tile - nfull * C
        for k in range(nfull):
            pltpu.sync_copy(zbuf, agg_sh.at[pl.ds(row0 + k * C, C)])
        if rem:
            pltpu.sync_copy(zbuf.at[pl.ds(0, rem)],
                            agg_sh.at[pl.ds(row0 + nfull * C, rem)])
        plsc.subcore_barrier()

        pltpu.sync_copy(src_hbm.at[c, s], src_v)
        pltpu.sync_copy(dst_hbm.at[s], dst_v)

        def gather(i, b):
            pltpu.async_copy(h_hbm.at[src_v.at[i]], rows_v.at[b], sem.at[b])

        gather(0, 0)
        gather(1, 1)

        def body(i, _):
            b = lax.rem(i, NB)

            @pl.when(i + 2 < CPW)
            def _():
                gather(i + 2, lax.rem(i + 2, NB))

            pltpu.make_async_copy(h_hbm.at[src_v.at[i]], rows_v.at[b],
                                  sem.at[b]).wait()
            pltpu.sync_copy(rows_v.at[b], agg_sh.at[dst_v.at[i]], add=True)
            return 0
        lax.fori_loop(0, CPW, body, 0)

        plsc.subcore_barrier()
        col0 = pl.multiple_of(c * DH, 8)
        pltpu.sync_copy(agg_sh.at[pl.ds(row0, rows_per_tile)],
                        agg_hbm.at[pl.ds(row0, rows_per_tile),
                                   pl.ds(col0, DH)])

    return edge_kernel


def _xs_body(N, x_ref, deg_ref, o_ref):
    NP = x_ref.shape[0]
    DH = D // NC
    dego = deg_ref[pl.ds(0, NP), :]
    norm = lax.rsqrt(jnp.maximum(dego[:, 0:1], 1.0))
    xs = x_ref[...] * norm
    o_ref[pl.ds(0, NP), :] = xs[:, :DH]
    o_ref[pl.ds(NP, NP), :] = xs[:, DH:]


def _layer_body(N, agg_ref, deg_ref, w_ref, b_ref, o_ref):
    NP = agg_ref.shape[0]
    DH = D // NC
    agg = agg_ref[...]
    degi = deg_ref[pl.ds(NP, NP), :]
    dego = deg_ref[pl.ds(0, NP), :]
    ni = lax.rsqrt(jnp.maximum(degi[:, 0:1], 1.0))
    no = lax.rsqrt(jnp.maximum(dego[:, 0:1], 1.0))
    h = jnp.dot(agg * ni, w_ref[...], preferred_element_type=jnp.float32)
    h = jnp.maximum(h + b_ref[...], 0.0)
    mask = lax.broadcasted_iota(jnp.int32, (NP, 1), 0) < N
    h = jnp.where(mask, h * no, 0.0)
    o_ref[pl.ds(0, NP), :] = h[:, :DH]
    o_ref[pl.ds(NP, NP), :] = h[:, DH:]


def _final_body(N, agg_ref, deg_ref, w_ref, b_ref,
                wd1_ref, bd1_ref, wd2_ref, bd2_ref, o_ref):
    NP = agg_ref.shape[0]
    agg = agg_ref[...]
    degi = deg_ref[pl.ds(NP, NP), :]
    ni = lax.rsqrt(jnp.maximum(degi[:, 0:1], 1.0))
    h = jnp.dot(agg * ni, w_ref[...], preferred_element_type=jnp.float32)
    h = jnp.maximum(h + b_ref[...], 0.0)
    mask = lax.broadcasted_iota(jnp.int32, (NP, 1), 0) < N
    h = jnp.where(mask, h, 0.0)
    g = jnp.sum(h, axis=0, keepdims=True)
    g = jnp.dot(g, wd1_ref[...], preferred_element_type=jnp.float32)
    g = jnp.maximum(g + bd1_ref[...], 0.0)
    g = jnp.dot(g, wd2_ref[...], preferred_element_type=jnp.float32)
    o_ref[...] = g + bd2_ref[...]


def kernel(x, edge_index, W0, b0, W1, b1, Wd1, bd1, Wd2, bd2):
    N = x.shape[0]
    E = edge_index.shape[1]
    NP = ((N + 2 * D) // (2 * D)) * (2 * D)
    CPW = -(-E // (NS * C))
    EP = NS * CPW * C

    src = edge_index[0]
    dst = edge_index[1]

    npad_rows = NP - N
    pad = N + (jnp.arange(EP - E, dtype=jnp.int32) % npad_rows)
    srcp = jnp.concatenate([src, pad]).reshape(NS, CPW, C)
    dstp = jnp.concatenate([dst, pad]).reshape(NS, CPW, C)
    degidx = jnp.concatenate([srcp, dstp + NP], axis=1)
    src2 = jnp.stack([srcp, srcp + NP])

    xp = jnp.pad(x, ((0, NP - N), (0, 0)))
    b0r, b1r = b0.reshape(1, D), b1.reshape(1, D)
    bd1r, bd2r = bd1.reshape(1, D), bd2.reshape(1, D)

    deg_kernel = _make_deg_kernel(NP, CPW)
    edge_kernel = _make_edge_kernel(NP, CPW)
    DH = D // NC

    deg = deg_kernel(degidx)

    xs = pl.pallas_call(
        functools.partial(_xs_body, N),
        out_shape=jax.ShapeDtypeStruct((NC * NP, DH), jnp.float32),
    )(xp, deg)

    agg1 = edge_kernel(xs, src2, dstp)

    h1s = pl.pallas_call(
        functools.partial(_layer_body, N),
        out_shape=jax.ShapeDtypeStruct((NC * NP, DH), jnp.float32),
    )(agg1, deg, W0, b0r)

    agg2 = edge_kernel(h1s, src2, dstp)

    out = pl.pallas_call(
        functools.partial(_final_body, N),
        out_shape=jax.ShapeDtypeStruct((1, D), jnp.float32),
    )(agg2, deg, W1, b1r, Wd1, bd1r, Wd2, bd2r)

    return out


# speedup vs baseline: 1.4851x; 1.0310x over previous
import functools

import jax
import jax.numpy as jnp
from jax import lax
from jax.experimental import pallas as pl
from jax.experimental.pallas import tpu as pltpu
from jax.experimental.pallas import tpu_sc as plsc

D = 128
L = 16
NC = 2
NS = 16
C = 128
DEGW = 16
IGN = -1


def _mesh():
    return plsc.VectorSubcoreMesh(
        core_axis_name="c", subcore_axis_name="s",
        num_cores=NC, num_subcores=NS)


def _make_deg_kernel(NP, CPW):
    rows_per_tile = NP // NS

    @functools.partial(
        pl.kernel,
        out_type=jax.ShapeDtypeStruct((NC * NP, DEGW), jnp.float32),
        mesh=_mesh(),
        scratch_types=[
            pltpu.VMEM((2 * CPW, C), jnp.int32),
            pltpu.VMEM((C, DEGW), jnp.float32),
            pltpu.VMEM((rows_per_tile, DEGW), jnp.float32),
            pltpu.VMEM_SHARED((NC * NP, DEGW), jnp.float32),
        ],
        compiler_params=pltpu.CompilerParams(use_tc_tiling_on_sc=False),
    )
    def deg_kernel(idx_hbm, deg_hbm, idx_v, ones_v, z_v, deg_sh):
        c = lax.axis_index("c")
        s = lax.axis_index("s")

        one16 = jnp.full((L,), 1.0, jnp.float32)
        zero16 = jnp.zeros((L,), jnp.float32)

        def fill_ones(r, _):
            ones_v[r, :] = one16
            return 0
        lax.fori_loop(0, C, fill_ones, 0)

        def fill_zeros(r, _):
            z_v[r, :] = zero16
            return 0
        lax.fori_loop(0, rows_per_tile, fill_zeros, 0)

        pltpu.sync_copy(z_v, deg_sh.at[pl.ds(s * rows_per_tile,
                                             rows_per_tile)])
        pltpu.sync_copy(z_v, deg_sh.at[pl.ds(NP + s * rows_per_tile,
                                             rows_per_tile)])
        plsc.subcore_barrier()

        pltpu.sync_copy(idx_hbm.at[s], idx_v)

        def body(i, _):
            pltpu.sync_copy(ones_v, deg_sh.at[idx_v.at[i]], add=True)
            return 0
        lax.fori_loop(c * CPW, (c + 1) * CPW, body, 0)

        plsc.subcore_barrier()
        row0 = c * NP + s * rows_per_tile
        pltpu.sync_copy(deg_sh.at[pl.ds(row0, rows_per_tile)],
                        deg_hbm.at[pl.ds(row0, rows_per_tile)])

    return deg_kernel


def _make_edge_kernel(NP, CPW):
    DH = D // NC
    rows_per_tile = NP // NS
    NB = 3

    @functools.partial(
        pl.kernel,
        out_type=(
            jax.ShapeDtypeStruct((NP, D), jnp.float32),
            jax.ShapeDtypeStruct((NC * NP, DH), jnp.float32),
        ),
        mesh=_mesh(),
        scratch_types=[
            pltpu.VMEM((CPW, C), jnp.int32),
            pltpu.VMEM((CPW, C), jnp.int32),
            pltpu.VMEM((NB, C, DH), jnp.float32),
            pltpu.VMEM_SHARED((NP, DH), jnp.float32),
            pltpu.SemaphoreType.DMA((NB,)),
        ],
        compiler_params=pltpu.CompilerParams(use_tc_tiling_on_sc=False),
    )
    def edge_kernel(h_hbm, src_hbm, dst_hbm, agg_hbm, flat_hbm,
                    src_v, dst_v, rows_v, agg_sh, sem):
        c = lax.axis_index("c")
        s = lax.axis_index("s")
        row0 = s * rows_per_tile
        col0 = pl.multiple_of(c * DH, 8)

        zero16 = jnp.zeros((L,), jnp.float32)

        def fill_zeros(r, _):
            for k in range(DH // L):
                rows_v[0, r, pl.ds(k * L, L)] = zero16
            return 0
        lax.fori_loop(0, C, fill_zeros, 0)

        zbuf = rows_v.at[0]
        nfull = rows_per_tile // C
        rem = rows_per_tile - nfull * C
        for k in range(nfull):
            pltpu.sync_copy(zbuf, agg_sh.at[pl.ds(row0 + k * C, C)])
        if rem:
            pltpu.sync_copy(zbuf.at[pl.ds(0, rem)],
                            agg_sh.at[pl.ds(row0 + nfull * C, rem)])

        for k in range(nfull):
            pltpu.sync_copy(
                h_hbm.at[pl.ds(row0 + k * C, C), pl.ds(col0, DH)],
                rows_v.at[1])
            pltpu.sync_copy(
                rows_v.at[1],
                flat_hbm.at[pl.ds(c * NP + row0 + k * C, C)])
        if rem:
            pltpu.sync_copy(
                h_hbm.at[pl.ds(row0 + nfull * C, rem), pl.ds(col0, DH)],
                rows_v.at[1, pl.ds(0, rem)])
            pltpu.sync_copy(
                rows_v.at[1, pl.ds(0, rem)],
                flat_hbm.at[pl.ds(c * NP + row0 + nfull * C, rem)])
        plsc.subcore_barrier()

        pltpu.sync_copy(src_hbm.at[c, s], src_v)
        pltpu.sync_copy(dst_hbm.at[s], dst_v)

        def gather(i, b):
            pltpu.async_copy(flat_hbm.at[src_v.at[i]], rows_v.at[b],
                             sem.at[b])

        gather(0, 0)
        gather(1, 1)

        def body(i, _):
            b = lax.rem(i, NB)

            @pl.when(i + 2 < CPW)
            def _():
                gather(i + 2, lax.rem(i + 2, NB))

            pltpu.make_async_copy(flat_hbm.at[src_v.at[i]], rows_v.at[b],
                                  sem.at[b]).wait()
            pltpu.sync_copy(rows_v.at[b], agg_sh.at[dst_v.at[i]], add=True)
            return 0
        lax.fori_loop(0, CPW, body, 0)

        plsc.subcore_barrier()
        col0 = pl.multiple_of(c * DH, 8)
        pltpu.sync_copy(agg_sh.at[pl.ds(row0, rows_per_tile)],
                        agg_hbm.at[pl.ds(row0, rows_per_tile),
                                   pl.ds(col0, DH)])

    return edge_kernel


def _xs_body(N, x_ref, deg_ref, o_ref):
    NP = x_ref.shape[0]
    dego = deg_ref[pl.ds(0, NP), :]
    norm = lax.rsqrt(jnp.maximum(dego[:, 0:1], 1.0))
    o_ref[...] = x_ref[...] * norm


def _layer_body(N, agg_ref, deg_ref, w_ref, b_ref, o_ref):
    NP = agg_ref.shape[0]
    DH = D // NC
    agg = agg_ref[...]
    degi = deg_ref[pl.ds(NP, NP), :]
    dego = deg_ref[pl.ds(0, NP), :]
    ni = lax.rsqrt(jnp.maximum(degi[:, 0:1], 1.0))
    no = lax.rsqrt(jnp.maximum(dego[:, 0:1], 1.0))
    h = jnp.dot(agg * ni, w_ref[...], preferred_element_type=jnp.float32)
    h = jnp.maximum(h + b_ref[...], 0.0)
    mask = lax.broadcasted_iota(jnp.int32, (NP, 1), 0) < N
    o_ref[...] = jnp.where(mask, h * no, 0.0)


def _final_body(N, agg_ref, deg_ref, w_ref, b_ref,
                wd1_ref, bd1_ref, wd2_ref, bd2_ref, o_ref):
    NP = agg_ref.shape[0]
    agg = agg_ref[...]
    degi = deg_ref[pl.ds(NP, NP), :]
    ni = lax.rsqrt(jnp.maximum(degi[:, 0:1], 1.0))
    h = jnp.dot(agg * ni, w_ref[...], preferred_element_type=jnp.float32)
    h = jnp.maximum(h + b_ref[...], 0.0)
    mask = lax.broadcasted_iota(jnp.int32, (NP, 1), 0) < N
    h = jnp.where(mask, h, 0.0)
    g = jnp.sum(h, axis=0, keepdims=True)
    g = jnp.dot(g, wd1_ref[...], preferred_element_type=jnp.float32)
    g = jnp.maximum(g + bd1_ref[...], 0.0)
    g = jnp.dot(g, wd2_ref[...], preferred_element_type=jnp.float32)
    o_ref[...] = g + bd2_ref[...]


def kernel(x, edge_index, W0, b0, W1, b1, Wd1, bd1, Wd2, bd2):
    N = x.shape[0]
    E = edge_index.shape[1]
    NP = ((N + 2 * D) // (2 * D)) * (2 * D)
    CPW = -(-E // (NS * C))
    EP = NS * CPW * C

    src = edge_index[0]
    dst = edge_index[1]

    npad_rows = NP - N
    pad = N + (jnp.arange(EP - E, dtype=jnp.int32) % npad_rows)
    srcp = jnp.concatenate([src, pad]).reshape(NS, CPW, C)
    dstp = jnp.concatenate([dst, pad]).reshape(NS, CPW, C)
    degidx = jnp.concatenate([srcp, dstp + NP], axis=1)
    src2 = jnp.stack([srcp, srcp + NP])

    xp = jnp.pad(x, ((0, NP - N), (0, 0)))
    b0r, b1r = b0.reshape(1, D), b1.reshape(1, D)
    bd1r, bd2r = bd1.reshape(1, D), bd2.reshape(1, D)

    deg_kernel = _make_deg_kernel(NP, CPW)
    edge_kernel = _make_edge_kernel(NP, CPW)
    DH = D // NC

    deg = deg_kernel(degidx)

    xs = pl.pallas_call(
        functools.partial(_xs_body, N),
        out_shape=jax.ShapeDtypeStruct((NP, D), jnp.float32),
    )(xp, deg)

    agg1, _ = edge_kernel(xs, src2, dstp)

    h1s = pl.pallas_call(
        functools.partial(_layer_body, N),
        out_shape=jax.ShapeDtypeStruct((NP, D), jnp.float32),
    )(agg1, deg, W0, b0r)

    agg2, _ = edge_kernel(h1s, src2, dstp)

    out = pl.pallas_call(
        functools.partial(_final_body, N),
        out_shape=jax.ShapeDtypeStruct((1, D), jnp.float32),
    )(agg2, deg, W1, b1r, Wd1, bd1r, Wd2, bd2r)

    return out
